# baseline (speedup 1.0000x reference)
import jax
import jax.numpy as jnp
from jax import lax
from jax.experimental import pallas as pl
from jax.experimental.pallas import tpu as pltpu
from jax.experimental.pallas import tpu_sc as plsc

B = 4096
N_NUM = 13
N_CAT = 26
SEQ_LEN = 50
VOCAB = 100000
D = 32
NCOLS = N_NUM + N_CAT + SEQ_LEN
NSLOT = N_NUM + N_CAT + 1
SUPW = 128
RPS = NSLOT * D // SUPW

NC, NS = 2, 16
NW = NC * NS
RPW = B // NW
C = 16
NCHUNK = RPW // C

CAT_SUP = N_CAT * C
CAT_H = CAT_SUP // 2
SEQ_SUP = SEQ_LEN * C
SEQ_H = SEQ_SUP // 2
OUT_SUP = C * RPS


def _fire_gather(tbl, idx_ref, base, n, buf, sem):
  descs = []
  off = 0
  while off < n:
    m = min(128, n - off)
    descs.append(pltpu.async_copy(
        tbl.at[idx_ref.at[pl.ds(base + off, m)]],
        buf.at[pl.ds(off, m)], sem))
    off += m
  return descs


def _body(x_hbm, w_hbm, cat_hbm, seq_hbm, out_hbm,
          xv, seq_v, cat_v, out_v, gidx_seq, gidx_cat, rcp_v, wv,
          sem_seq, sem_cat, sem_out):
  wid = lax.axis_index("s") * NC + lax.axis_index("c")
  lane = lax.iota(jnp.int32, 16)

  pltpu.sync_copy(w_hbm, wv)

  def chunk_body(ci, carry):
    gbase = wid * RPW + ci * C

    pltpu.sync_copy(x_hbm.at[pl.ds(gbase * NCOLS, C * NCOLS)], xv)

    rows89 = lane * NCOLS

    for j in range(N_CAT):
      v = plsc.load_gather(xv, [rows89 + (N_NUM + j)])
      gidx_cat[pl.ds(j * C, C)] = (
          j * (VOCAB // 4) + lax.shift_right_logical(v, 2))
    cnt = jnp.zeros((16,), jnp.float32)
    for t in range(SEQ_LEN):
      v = plsc.load_gather(xv, [rows89 + (N_NUM + N_CAT + t)])
      gidx_seq[pl.ds(t * C, C)] = lax.shift_right_logical(v, 2)
      cnt = cnt + jnp.where(v != 0, 1.0, 0.0)
    rcp_v[pl.ds(0, 16)] = 1.0 / jnp.maximum(cnt, 1e-12)

    seq_descs = _fire_gather(seq_hbm, gidx_seq, 0, SEQ_H, seq_v, sem_seq)
    cat_descs = _fire_gather(cat_hbm, gidx_cat, 0, CAT_H, cat_v, sem_cat)

    for dsc in seq_descs:
      dsc.wait()

    def nrow_body(b, c2):
      xoff = b * NCOLS
      orow = b * RPS
      for i in range(N_NUM):
        xi = plsc.load_gather(
            xv, [jnp.full((16,), xoff + i, jnp.int32)]).astype(jnp.float32)
        r = orow + (i * D) // SUPW
        c0 = (i * D) % SUPW
        out_v[r, pl.ds(c0, 16)] = xi * wv[pl.ds(i * D, 16)]
        out_v[r, pl.ds(c0 + 16, 16)] = xi * wv[pl.ds(i * D + 16, 16)]
      acc0 = jnp.zeros((16,), jnp.float32)
      acc1 = jnp.zeros((16,), jnp.float32)
      for t in range(SEQ_LEN // 2):
        xq = plsc.load_gather(
            xv, [jnp.full((16,), xoff + N_NUM + N_CAT + t, jnp.int32)])
        cq = lax.bitwise_and(xq, 3) * D + lane
        rr = jnp.full((16,), t * C + b, jnp.int32)
        acc0 = acc0 + plsc.load_gather(seq_v, [rr, cq])
        acc1 = acc1 + plsc.load_gather(seq_v, [rr, cq + 16])
      out_v[orow + RPS - 1, pl.ds(96, 16)] = acc0
      out_v[orow + RPS - 1, pl.ds(112, 16)] = acc1
      return c2

    lax.fori_loop(0, C, nrow_body, 0)

    seq_descs = _fire_gather(seq_hbm, gidx_seq, SEQ_H, SEQ_H, seq_v, sem_seq)

    for dsc in cat_descs:
      dsc.wait()

    def cat_extract(j0, b, _):
      for j in range(j0, j0 + N_CAT // 2):
        slot = N_NUM + j
        row = (j - j0) * C + b
        xq = plsc.load_gather(
            xv, [jnp.full((16,), b * NCOLS + N_NUM + j, jnp.int32)])
        cq = lax.bitwise_and(xq, 3) * D + lane
        rr = jnp.full((16,), row, jnp.int32)
        r = b * RPS + (slot * D) // SUPW
        c0 = (slot * D) % SUPW
        out_v[r, pl.ds(c0, 16)] = plsc.load_gather(cat_v, [rr, cq])
        out_v[r, pl.ds(c0 + 16, 16)] = plsc.load_gather(cat_v, [rr, cq + 16])
      return _

    lax.fori_loop(0, C, lambda b, c2: cat_extract(0, b, c2), 0)

    cat_descs = _fire_gather(cat_hbm, gidx_cat, CAT_H, CAT_H, cat_v, sem_cat)

    for dsc in seq_descs:
      dsc.wait()

    def srow_body(b, c2):
      xoff = b * NCOLS
      acc0 = jnp.zeros((16,), jnp.float32)
      acc1 = jnp.zeros((16,), jnp.float32)
      for t in range(SEQ_LEN // 2, SEQ_LEN):
        xq = plsc.load_gather(
            xv, [jnp.full((16,), xoff + N_NUM + N_CAT + t, jnp.int32)])
        cq = lax.bitwise_and(xq, 3) * D + lane
        rr = jnp.full((16,), (t - SEQ_LEN // 2) * C + b, jnp.int32)
        acc0 = acc0 + plsc.load_gather(seq_v, [rr, cq])
        acc1 = acc1 + plsc.load_gather(seq_v, [rr, cq + 16])
      rcp = plsc.load_gather(rcp_v, [jnp.full((16,), b, jnp.int32)])
      r = b * RPS + RPS - 1
      out_v[r, pl.ds(96, 16)] = (out_v[r, pl.ds(96, 16)] + acc0) * rcp
      out_v[r, pl.ds(112, 16)] = (out_v[r, pl.ds(112, 16)] + acc1) * rcp
      return c2

    lax.fori_loop(0, C, srow_body, 0)

    for dsc in cat_descs:
      dsc.wait()
    lax.fori_loop(0, C, lambda b, c2: cat_extract(N_CAT // 2, b, c2), 0)

    pltpu.async_copy(out_v, out_hbm.at[pl.ds(gbase * RPS, OUT_SUP)],
                     sem_out).wait()
    return carry

  lax.fori_loop(0, NCHUNK, chunk_body, 0)


_sc_call = pl.kernel(
    _body,
    out_type=jax.ShapeDtypeStruct((B * RPS, SUPW), jnp.float32),
    mesh=plsc.VectorSubcoreMesh(core_axis_name="c", subcore_axis_name="s"),
    compiler_params=pltpu.CompilerParams(
        needs_layout_passes=False, use_tc_tiling_on_sc=True),
    scratch_types=[
        pltpu.VMEM((C * NCOLS,), jnp.int32),
        pltpu.VMEM((SEQ_H, SUPW), jnp.float32),
        pltpu.VMEM((CAT_H, SUPW), jnp.float32),
        pltpu.VMEM((OUT_SUP, SUPW), jnp.float32),
        pltpu.VMEM((SEQ_SUP,), jnp.int32),
        pltpu.VMEM((CAT_SUP,), jnp.int32),
        pltpu.VMEM((16,), jnp.float32),
        pltpu.VMEM((N_NUM * D,), jnp.float32),
        pltpu.SemaphoreType.DMA,
        pltpu.SemaphoreType.DMA,
        pltpu.SemaphoreType.DMA,
    ],
)


@jax.jit
def kernel(X, W_num, cat_tables, seq_table):
  cat_sup = cat_tables.reshape(N_CAT * VOCAB // 4, SUPW)
  seq_sup = seq_table.reshape(VOCAB // 4, SUPW)
  out = _sc_call(X.reshape(B * NCOLS), W_num.reshape(N_NUM * D),
                 cat_sup, seq_sup)
  return out.reshape(B, NSLOT, D)
